# baseline (speedup 1.0000x reference)
import jax
import jax.numpy as jnp
from jax.experimental import pallas as pl


def _pos_body(w_row_ref, w_col_ref, out_ref):
    out_ref[...] = w_row_ref[...][:, None, :] + w_col_ref[...][None, :, :]


def kernel(input, W_row, W_col):
    h, w = input.shape[1], input.shape[2]
    d = W_row.shape[1]
    bh = 8
    grid = (h // bh,)
    return pl.pallas_call(
        _pos_body,
        grid=grid,
        in_specs=[
            pl.BlockSpec((bh, d), lambda i: (i, 0)),
            pl.BlockSpec((w, d), lambda i: (0, 0)),
        ],
        out_specs=pl.BlockSpec((bh, w, d), lambda i: (i, 0, 0)),
        out_shape=jax.ShapeDtypeStruct((h, w, d), jnp.float32),
    )(W_row[:h], W_col[:w])
